# baseline (speedup 1.0000x reference)
import functools

import jax
import jax.numpy as jnp
from jax import lax
from jax.experimental import pallas as pl
from jax.experimental.pallas import tpu as pltpu
from jax.experimental.pallas import tpu_sc as plsc

_N = 16384
_C = 26
_K = 101
_W = _C * _K
_NW = 32
_RPW = _N // _NW
_G = 16
_GW = _G * _W
_NBUF = 2
_NPAIR = _RPW // (_G * _NBUF)


def _sc_body(x_hbm, m_hbm, out_hbm, xv, mv, buf0, buf1, sem0, sem1):
    wid = lax.axis_index("s") * 2 + lax.axis_index("c")
    wbase = wid * _RPW
    pltpu.sync_copy(x_hbm.at[pl.ds(wbase * 32, _RPW * 32)], xv)
    pltpu.sync_copy(m_hbm.at[pl.ds(wbase * 32, _RPW * 32)], mv)

    bufs = (buf0, buf1)
    sems = (sem0, sem1)
    iota = lax.iota(jnp.int32, 16)
    cat0 = iota * _K
    cat1 = (iota + 16) * _K
    msk1 = iota < (_C - 16)
    ones = jnp.full((16,), 1.0, jnp.float32)
    zeros = jnp.zeros((16,), jnp.float32)

    def _zfill(k, _):
        cols = k * 16 + iota
        plsc.store_scatter(buf0, [cols], zeros)
        plsc.store_scatter(buf1, [cols], zeros)
        return _

    lax.fori_loop(0, _GW // 16, _zfill, 0)

    def _positions(lr, rbase):
        xs0 = xv[pl.ds(lr * 32, 16)]
        ms0 = mv[pl.ds(lr * 32, 16)]
        xs1 = xv[pl.ds(lr * 32 + 16, 16)]
        ms1 = mv[pl.ds(lr * 32 + 16, 16)]
        p0 = ((xs0 + 1.0) * ms0).astype(jnp.int32) + cat0 + rbase
        p1 = ((xs1 + 1.0) * ms1).astype(jnp.int32) + cat1 + rbase
        return p0, p1

    def _group(i, _):
        for b in range(_NBUF):
            g = i * _NBUF + b
            buf = bufs[b]
            sem = sems[b]

            @pl.when(i > 0)
            def _reuse(g=g, buf=buf, sem=sem):
                pltpu.make_async_copy(out_hbm.at[pl.ds(0, _GW)], buf, sem).wait()
                gp = g - _NBUF
                for r in range(_G):
                    p0, p1 = _positions(gp * _G + r, r * _W)
                    plsc.store_scatter(buf, [p0], zeros)
                    plsc.store_scatter(buf, [p1], zeros, mask=msk1)

            for r in range(_G):
                p0, p1 = _positions(g * _G + r, r * _W)
                plsc.store_scatter(buf, [p0], ones)
                plsc.store_scatter(buf, [p1], ones, mask=msk1)
            pltpu.make_async_copy(
                buf, out_hbm.at[pl.ds((wbase + g * _G) * _W, _GW)], sem
            ).start()
        return _

    lax.fori_loop(0, _NPAIR, _group, 0)

    for b in range(_NBUF):
        pltpu.make_async_copy(out_hbm.at[pl.ds(0, _GW)], bufs[b], sems[b]).wait()


@functools.lru_cache(maxsize=1)
def _build():
    mesh = plsc.VectorSubcoreMesh(core_axis_name="c", subcore_axis_name="s")
    return pl.kernel(
        _sc_body,
        mesh=mesh,
        compiler_params=pltpu.CompilerParams(needs_layout_passes=False),
        out_type=jax.ShapeDtypeStruct((_N * _W,), jnp.float32),
        scratch_types=[
            pltpu.VMEM((_RPW * 32,), jnp.float32),
            pltpu.VMEM((_RPW * 32,), jnp.float32),
            pltpu.VMEM((_GW,), jnp.float32),
            pltpu.VMEM((_GW,), jnp.float32),
            pltpu.SemaphoreType.DMA,
            pltpu.SemaphoreType.DMA,
        ],
    )


def kernel(x, mask):
    pad = ((0, 0), (0, 32 - _C))
    x32 = jnp.pad(x, pad).reshape(-1)
    m32 = jnp.pad(mask, pad).reshape(-1)
    return _build()(x32, m32).reshape(_N, _W)


# speedup vs baseline: 2.3773x; 2.3773x over previous
import functools

import jax
import jax.numpy as jnp
import numpy as np
from jax.experimental import pallas as pl
from jax.experimental.pallas import tpu as pltpu

_N = 16384
_C = 26
_K = 101
_W = _C * _K
_BLOCK = 512
_NBLK = _N // _BLOCK
_NBUF = 4


def _onehot_kernel(x_ref, m_ref, s_ref, k_ref, o_ref, *rest):
    bufs = rest[:_NBUF]
    sems = rest[_NBUF:]
    idx = (x_ref[...] + 1.0) * m_ref[...]
    kvec = k_ref[...]
    sel = s_ref[...]
    for i in range(_NBLK):
        b = i % _NBUF
        if i >= _NBUF:
            pltpu.make_async_copy(
                bufs[b], o_ref.at[pl.ds((i - _NBUF) * _BLOCK, _BLOCK)], sems[b]
            ).wait()
        spread = jnp.dot(
            idx[i * _BLOCK:(i + 1) * _BLOCK, :],
            sel,
            preferred_element_type=jnp.float32,
        )
        bufs[b][...] = (spread == kvec).astype(jnp.float32)
        pltpu.make_async_copy(
            bufs[b], o_ref.at[pl.ds(i * _BLOCK, _BLOCK)], sems[b]
        ).start()
    for i in range(_NBLK - _NBUF, _NBLK):
        b = i % _NBUF
        pltpu.make_async_copy(
            bufs[b], o_ref.at[pl.ds(i * _BLOCK, _BLOCK)], sems[b]
        ).wait()


@functools.lru_cache(maxsize=1)
def _constants():
    j = np.arange(_W)
    sel = (j // _K == np.arange(_C)[:, None]).astype(np.float32)
    kvec = (j % _K).astype(np.float32).reshape(1, _W)
    return jnp.asarray(sel), jnp.asarray(kvec)


def kernel(x, mask):
    n, c = x.shape
    sel, kvec = _constants()
    scratch = [pltpu.VMEM((_BLOCK, _W), jnp.float32) for _ in range(_NBUF)]
    scratch += [pltpu.SemaphoreType.DMA for _ in range(_NBUF)]
    return pl.pallas_call(
        _onehot_kernel,
        in_specs=[
            pl.BlockSpec(memory_space=pltpu.VMEM),
            pl.BlockSpec(memory_space=pltpu.VMEM),
            pl.BlockSpec(memory_space=pltpu.VMEM),
            pl.BlockSpec(memory_space=pltpu.VMEM),
        ],
        out_specs=pl.BlockSpec(memory_space=pltpu.HBM),
        out_shape=jax.ShapeDtypeStruct((n, _W), jnp.float32),
        scratch_shapes=scratch,
    )(x, mask, sel, kvec)
